# baseline (speedup 1.0000x reference)
import jax
import jax.numpy as jnp
from jax import lax
from jax.experimental import pallas as pl
from jax.experimental.pallas import tpu as pltpu
from jax.experimental.pallas import tpu_sc as plsc

NUM_SEG = 33
F = 128
N = 100000
NC, NS, L = 2, 16, 16
NW = NC * NS
CH = 400
NCH = 8
BIGW = 10
NBIG = 3136
NSML = 3120
ACC = NUM_SEG * F


def _sc_body(emb_hbm, dep_hbm, sums_hbm, maxs_hbm, cnts_hbm,
             dep_v, chunk_v, sum_v, max_v, cnt_v):
    wid = lax.axis_index("s") * NC + lax.axis_index("c")
    is_big = wid < BIGW
    n_loc = jnp.where(is_big, NBIG, NSML)
    start = jnp.where(is_big, wid * NBIG, BIGW * NBIG + (wid - BIGW) * NSML)

    @pl.when(is_big)
    def _():
        pltpu.sync_copy(dep_hbm.at[pl.ds(start, NBIG)], dep_v.at[pl.ds(0, NBIG)])

    @pl.when(jnp.logical_not(is_big))
    def _():
        pltpu.sync_copy(dep_hbm.at[pl.ds(start, NSML)], dep_v.at[pl.ds(0, NSML)])

    zeros = jnp.zeros((L,), jnp.float32)
    ninf = jnp.full((L,), -jnp.inf, jnp.float32)
    ones = jnp.full((L,), 1.0, jnp.float32)

    def _init(i, c):
        sum_v[pl.ds(i * L, L)] = zeros
        max_v[pl.ds(i * L, L)] = ninf
        return c

    lax.fori_loop(0, ACC // L, _init, 0)

    def _initc(i, c):
        cnt_v[pl.ds(i * L, L)] = zeros
        return c

    lax.fori_loop(0, NUM_SEG, _initc, 0)

    for k in range(NCH):
        if k < NCH - 1:
            o_k = start + k * CH
            base_l = k * CH
            row_lo = 0
        else:
            o_k = start + n_loc - CH
            base_l = n_loc - CH
            row_lo = NCH * CH - n_loc

        pltpu.sync_copy(emb_hbm.at[pl.ds(o_k, CH)], chunk_v)

        def _row(r, c, base_l=base_l):
            d = dep_v[pl.ds(base_l + r, L)][0]
            a = d * F
            for j in range(F // L):
                off = a + j * L
                v = chunk_v[r, pl.ds(j * L, L)]
                sum_v[pl.ds(off, L)] = sum_v[pl.ds(off, L)] + v
                max_v[pl.ds(off, L)] = jnp.maximum(max_v[pl.ds(off, L)], v)
            co = d * L
            cnt_v[pl.ds(co, L)] = cnt_v[pl.ds(co, L)] + ones
            return c

        lax.fori_loop(row_lo, CH, _row, 0)

    pltpu.sync_copy(sum_v, sums_hbm.at[wid])
    pltpu.sync_copy(max_v, maxs_hbm.at[wid])
    pltpu.sync_copy(cnt_v, cnts_hbm.at[wid])


def _merge_body(sums_ref, maxs_ref, cnts_ref, out_ref):
    s = jnp.sum(sums_ref[...], axis=0)
    m = jnp.max(maxs_ref[...], axis=0)
    c = jnp.sum(cnts_ref[...], axis=0)[:, :1]
    mean = s / jnp.maximum(c, 1.0)
    ne = c > 0.0
    out_ref[:, :F] = jnp.where(ne, mean, 0.0)
    out_ref[:, F:] = jnp.where(ne, m, 0.0)


def kernel(node_embeddings, node_depths, max_depth):
    dep = jnp.minimum(node_depths, max_depth).astype(jnp.int32)

    mesh = plsc.VectorSubcoreMesh(core_axis_name="c", subcore_axis_name="s")
    sums, maxs, cnts = pl.kernel(
        _sc_body,
        out_type=(
            jax.ShapeDtypeStruct((NW, ACC), jnp.float32),
            jax.ShapeDtypeStruct((NW, ACC), jnp.float32),
            jax.ShapeDtypeStruct((NW, NUM_SEG * L), jnp.float32),
        ),
        mesh=mesh,
        scratch_types=[
            pltpu.VMEM((NBIG + L,), jnp.int32),
            pltpu.VMEM((CH, F), jnp.float32),
            pltpu.VMEM((ACC,), jnp.float32),
            pltpu.VMEM((ACC,), jnp.float32),
            pltpu.VMEM((NUM_SEG * L,), jnp.float32),
        ],
    )(node_embeddings, dep)

    out = pl.pallas_call(
        _merge_body,
        out_shape=jax.ShapeDtypeStruct((NUM_SEG, 2 * F), jnp.float32),
    )(
        sums.reshape(NW, NUM_SEG, F),
        maxs.reshape(NW, NUM_SEG, F),
        cnts.reshape(NW, NUM_SEG, L),
    )
    return out


# speedup vs baseline: 3.7357x; 3.7357x over previous
import jax
import jax.numpy as jnp
from jax import lax
from jax.experimental import pallas as pl
from jax.experimental.pallas import tpu as pltpu
from jax.experimental.pallas import tpu_sc as plsc

NUM_SEG = 33
F = 128
NV = F // 16
N = 100000
NC, NS, L = 2, 16, 16
NW = NC * NS
CH = 400
NCH = 8
BIGW = 10
NBIG = 3136
NSML = 3120
ACC = NUM_SEG * F


def _sc_body(emb_hbm, dep_hbm, sums_hbm, maxs_hbm, cnts_hbm,
             dep_v, buf0, buf1, sum_v, max_v, cnt_v, bnd_s, sem0, sem1):
    wid = lax.axis_index("s") * NC + lax.axis_index("c")
    is_big = wid < BIGW
    n_loc = jnp.where(is_big, NBIG, NSML)
    start = jnp.where(is_big, wid * NBIG, BIGW * NBIG + (wid - BIGW) * NSML)

    @pl.when(is_big)
    def _():
        pltpu.sync_copy(dep_hbm.at[pl.ds(start, NBIG)], dep_v.at[pl.ds(0, NBIG)])

    @pl.when(jnp.logical_not(is_big))
    def _():
        pltpu.sync_copy(dep_hbm.at[pl.ds(start, NSML)], dep_v.at[pl.ds(0, NSML)])

    zeros = jnp.zeros((L,), jnp.float32)
    ninf = jnp.full((L,), -jnp.inf, jnp.float32)

    def _bnd(d, c):
        def _step(_, lohi):
            lo, hi = lohi
            mid = lax.shift_right_logical(lo + hi, 1)
            v = dep_v[pl.ds(mid, L)][0]
            act = lo < hi
            p = act & (v < d)
            return (jnp.where(p, mid + 1, lo),
                    jnp.where(act & jnp.logical_not(p), mid, hi))
        lo, _ = lax.fori_loop(0, 12, _step, (0, n_loc))
        bnd_s[d] = lo
        return c

    lax.fori_loop(0, NUM_SEG + 1, _bnd, 0)

    def _cnt(d, c):
        cw = (bnd_s[d + 1] - bnd_s[d]).astype(jnp.float32)
        cnt_v[pl.ds(d * L, L)] = jnp.full((L,), cw, jnp.float32)
        return c

    lax.fori_loop(0, NUM_SEG, _cnt, 0)

    def _init(i, c):
        sum_v[pl.ds(i * L, L)] = zeros
        max_v[pl.ds(i * L, L)] = ninf
        return c

    lax.fori_loop(0, ACC // L, _init, 0)

    bufs = (buf0, buf1)
    sems = (sem0, sem1)

    def _mk_copy(k):
        o_k = start + (k * CH if k < NCH - 1 else n_loc - CH)
        return pltpu.make_async_copy(
            emb_hbm.at[pl.ds(o_k, CH)], bufs[k % 2], sems[k % 2])

    _mk_copy(0).start()
    for k in range(NCH):
        if k + 1 < NCH:
            _mk_copy(k + 1).start()
        _mk_copy(k).wait()
        buf = bufs[k % 2]

        if k < NCH - 1:
            base_l = k * CH
            p_lo = k * CH
            p_hi = (k + 1) * CH
        else:
            base_l = n_loc - CH
            p_lo = (NCH - 1) * CH
            p_hi = n_loc

        d_first = dep_v[pl.ds(p_lo, L)][0]
        d_last = dep_v[pl.ds(p_hi - 1, L)][0]

        def _seg(d, c, base_l=base_l, p_lo=p_lo, p_hi=p_hi, buf=buf):
            lo = jnp.maximum(bnd_s[d], p_lo) - base_l
            hi = jnp.minimum(bnd_s[d + 1], p_hi) - base_l

            def _row(r, carry):
                out = []
                for j in range(NV):
                    v = buf[r, pl.ds(j * L, L)]
                    out.append(carry[j] + v)
                for j in range(NV):
                    v = buf[r, pl.ds(j * L, L)]
                    out.append(jnp.maximum(carry[NV + j], v))
                return tuple(out)

            carry = lax.fori_loop(lo, hi, _row, (zeros,) * NV + (ninf,) * NV)

            @pl.when(hi > lo)
            def _merge():
                a = d * F
                for j in range(NV):
                    off = a + j * L
                    sum_v[pl.ds(off, L)] = sum_v[pl.ds(off, L)] + carry[j]
                    max_v[pl.ds(off, L)] = jnp.maximum(
                        max_v[pl.ds(off, L)], carry[NV + j])
            return c

        lax.fori_loop(d_first, d_last + 1, _seg, 0)

    pltpu.sync_copy(sum_v, sums_hbm.at[wid])
    pltpu.sync_copy(max_v, maxs_hbm.at[wid])
    pltpu.sync_copy(cnt_v, cnts_hbm.at[wid])


def _merge_body(sums_ref, maxs_ref, cnts_ref, out_ref):
    s = jnp.sum(sums_ref[...], axis=0)
    m = jnp.max(maxs_ref[...], axis=0)
    c = jnp.sum(cnts_ref[...], axis=0)[:, :1]
    mean = s / jnp.maximum(c, 1.0)
    ne = c > 0.0
    out_ref[:, :F] = jnp.where(ne, mean, 0.0)
    out_ref[:, F:] = jnp.where(ne, m, 0.0)


def kernel(node_embeddings, node_depths, max_depth):
    dep = jnp.minimum(node_depths, max_depth).astype(jnp.int32)

    mesh = plsc.VectorSubcoreMesh(core_axis_name="c", subcore_axis_name="s")
    sums, maxs, cnts = pl.kernel(
        _sc_body,
        out_type=(
            jax.ShapeDtypeStruct((NW, ACC), jnp.float32),
            jax.ShapeDtypeStruct((NW, ACC), jnp.float32),
            jax.ShapeDtypeStruct((NW, NUM_SEG * L), jnp.float32),
        ),
        mesh=mesh,
        scratch_types=[
            pltpu.VMEM((NBIG + L,), jnp.int32),
            pltpu.VMEM((CH, F), jnp.float32),
            pltpu.VMEM((CH, F), jnp.float32),
            pltpu.VMEM((ACC,), jnp.float32),
            pltpu.VMEM((ACC,), jnp.float32),
            pltpu.VMEM((NUM_SEG * L,), jnp.float32),
            pltpu.SMEM((NUM_SEG + 1,), jnp.int32),
            pltpu.SemaphoreType.DMA,
            pltpu.SemaphoreType.DMA,
        ],
    )(node_embeddings, dep)

    out = pl.pallas_call(
        _merge_body,
        out_shape=jax.ShapeDtypeStruct((NUM_SEG, 2 * F), jnp.float32),
    )(
        sums.reshape(NW, NUM_SEG, F),
        maxs.reshape(NW, NUM_SEG, F),
        cnts.reshape(NW, NUM_SEG, L),
    )
    return out
